# initial kernel scaffold (unmeasured)
import functools

import jax
import jax.numpy as jnp
from jax import lax
from jax.experimental import pallas as pl
from jax.experimental.pallas import tpu as pltpu

N_DEV = 8
HEADS = 8
DH = 128
SQ = 1024
SKV = 1024
D = 1024
SCALE = 0.08838834764831843
NEG = -1e9


def _body(x_ref, wq_ref, k_ref, v_ref, wo_ref, out_ref,
          comm_ref, q_ref, ctx_ref, mask_ref,
          send_sems, recv_sems, credit_sem):
    my = lax.axis_index("i")
    left = jnp.mod(my - 1, N_DEV)
    right = jnp.mod(my + 1, N_DEV)

    barrier = pltpu.get_barrier_semaphore()
    for nbr in (left, right):
        pl.semaphore_signal(barrier, inc=1, device_id=(nbr,),
                            device_id_type=pl.DeviceIdType.MESH)
    pl.semaphore_wait(barrier, 2)

    qb = lax.broadcasted_iota(jnp.int32, (SQ, SKV), 0) // 64
    kb = lax.broadcasted_iota(jnp.int32, (SQ, SKV), 1) // 64
    keep = (qb == kb) | (kb == 0) | (jnp.mod(qb + kb, 3) == 0)
    mask_ref[...] = jnp.where(keep, 0.0, NEG).astype(jnp.float32)

    comm_ref[0, pl.ds(0, D), :] = wq_ref[...]
    comm_ref[0, pl.ds(D, D), :] = wo_ref[...]
    out_ref[0, :, :] = jnp.zeros((SQ, D), jnp.float32)

    for h in range(N_DEV):
        slot = h % 2
        g = jnp.mod(my - h, N_DEV)

        rdma = None
        if h < N_DEV - 1:
            if h >= 1:
                pl.semaphore_wait(credit_sem, 1)
            rdma = pltpu.make_async_remote_copy(
                src_ref=comm_ref.at[slot],
                dst_ref=comm_ref.at[1 - slot],
                send_sem=send_sems.at[slot],
                recv_sem=recv_sems.at[1 - slot],
                device_id=(right,),
                device_id_type=pl.DeviceIdType.MESH,
            )
            rdma.start()

        wq_g = comm_ref[slot, pl.ds(0, D), :]
        wo_g = comm_ref[slot, pl.ds(D, D), :]
        q_ref[...] = jax.lax.dot(
            x_ref[...], wq_g, preferred_element_type=jnp.float32
        ).astype(jnp.bfloat16)

        def head_body(hh, carry, g=g):
            off = hh * DH
            gh = g * HEADS + hh
            q_h = q_ref[:, pl.ds(off, DH)]
            k_h = k_ref[pl.ds(gh, 1)][0]
            s = lax.dot_general(
                q_h, k_h, (((1,), (1,)), ((), ())),
                preferred_element_type=jnp.float32,
            )
            s = s * SCALE + mask_ref[...]
            m = jnp.max(s, axis=1, keepdims=True)
            w = jnp.exp(s - m)
            w = (w / jnp.sum(w, axis=1, keepdims=True)).astype(jnp.bfloat16)
            v_h = v_ref[pl.ds(gh, 1)][0]
            c = lax.dot_general(
                w, v_h, (((1,), (0,)), ((), ())),
                preferred_element_type=jnp.float32,
            )
            ctx_ref[:, pl.ds(off, DH)] = c.astype(jnp.bfloat16)
            return carry

        lax.fori_loop(0, HEADS, head_body, 0)

        out_ref[0, :, :] = out_ref[0, :, :] + jax.lax.dot(
            ctx_ref[...], wo_g, preferred_element_type=jnp.float32
        )

        if h <= N_DEV - 3:
            pl.semaphore_signal(credit_sem, inc=1, device_id=(left,),
                                device_id_type=pl.DeviceIdType.MESH)
        if rdma is not None:
            rdma.wait()

    @functools.partial(pl.run_scoped, exit_sem=pltpu.SemaphoreType.REGULAR)
    def _(exit_sem):
        for nbr in (left, right):
            pl.semaphore_signal(exit_sem, inc=1, device_id=(nbr,),
                                device_id_type=pl.DeviceIdType.MESH)
        pl.semaphore_wait(exit_sem, 2)


def kernel(x, Wq, K_ext, V_ext, Wo):
    i = lax.axis_index("i")
    xs = x[0].astype(jnp.bfloat16)
    wq = Wq.astype(jnp.bfloat16)
    wo = Wo.astype(jnp.bfloat16)
    k = lax.dynamic_index_in_dim(K_ext, i, 0, keepdims=False)
    v = lax.dynamic_index_in_dim(V_ext, i, 0, keepdims=False)
    kt = jnp.transpose(k, (1, 0, 2)).astype(jnp.bfloat16)
    vt = jnp.transpose(v, (1, 0, 2)).astype(jnp.bfloat16)

    vmem = functools.partial(pl.BlockSpec, memory_space=pltpu.VMEM)
    return pl.pallas_call(
        _body,
        out_shape=jax.ShapeDtypeStruct((1, SQ, D), jnp.float32),
        in_specs=[vmem(), vmem(), vmem(), vmem(), vmem()],
        out_specs=vmem(),
        scratch_shapes=[
            pltpu.VMEM((2, 2 * D, D), jnp.bfloat16),
            pltpu.VMEM((SQ, HEADS * DH), jnp.bfloat16),
            pltpu.VMEM((SQ, HEADS * DH), jnp.bfloat16),
            pltpu.VMEM((SQ, SKV), jnp.float32),
            pltpu.SemaphoreType.DMA((2,)),
            pltpu.SemaphoreType.DMA((2,)),
            pltpu.SemaphoreType.REGULAR,
        ],
        compiler_params=pltpu.CompilerParams(
            collective_id=0,
            vmem_limit_bytes=128 * 1024 * 1024,
        ),
    )(xs, wq, kt, vt, wo)


# baseline (device time: 459655 ns/iter reference)
import functools
import os

import jax

os.makedirs("/tmp/jax_cache", exist_ok=True)
jax.config.update("jax_compilation_cache_dir", "/tmp/jax_cache")
jax.config.update("jax_persistent_cache_min_compile_time_secs", 0.0)
jax.config.update("jax_persistent_cache_min_entry_size_bytes", -1)

import jax.numpy as jnp
from jax import lax
from jax.experimental import pallas as pl
from jax.experimental.pallas import tpu as pltpu

N_DEV = 8
HEADS = 8
DH = 128
SQ = 1024
SKV = 1024
D = 1024
SCALE = 0.08838834764831843
NEG = -1e9


def _body(x_ref, wq_ref, k_ref, v_ref, wo_ref, out_ref,
          comm_ref, q_ref, ctx_ref, mask_ref, kbuf, vbuf,
          send_sems, recv_sems, kv_sems, credit_sem):
    my = lax.axis_index("i")
    left = jnp.mod(my - 1, N_DEV)
    right = jnp.mod(my + 1, N_DEV)

    def start_kv(slot, g):
        kc = pltpu.make_async_copy(
            k_ref.at[pl.ds(g * HEADS, HEADS)], kbuf.at[slot],
            kv_sems.at[slot, 0])
        vc = pltpu.make_async_copy(
            v_ref.at[pl.ds(g * HEADS, HEADS)], vbuf.at[slot],
            kv_sems.at[slot, 1])
        kc.start()
        vc.start()
        return kc, vc

    kv = start_kv(0, my)

    barrier = pltpu.get_barrier_semaphore()
    for nbr in (left, right):
        pl.semaphore_signal(barrier, inc=1, device_id=(nbr,),
                            device_id_type=pl.DeviceIdType.MESH)
    pl.semaphore_wait(barrier, 2)

    qb = lax.broadcasted_iota(jnp.int32, (SQ, SKV), 0) // 64
    kb = lax.broadcasted_iota(jnp.int32, (SQ, SKV), 1) // 64
    keep = (qb == kb) | (kb == 0) | (jnp.mod(qb + kb, 3) == 0)
    mask_ref[...] = jnp.where(keep, 0.0, NEG).astype(jnp.float32)

    comm_ref[0, pl.ds(0, D), :] = wq_ref[...]
    comm_ref[0, pl.ds(D, D), :] = wo_ref[...]
    out_ref[0, :, :] = jnp.zeros((SQ, D), jnp.float32)

    for h in range(N_DEV):
        slot = h % 2
        g = jnp.mod(my - h, N_DEV)

        rdma = None
        if h < N_DEV - 1:
            if h >= 1:
                pl.semaphore_wait(credit_sem, 1)
            rdma = pltpu.make_async_remote_copy(
                src_ref=comm_ref.at[slot],
                dst_ref=comm_ref.at[1 - slot],
                send_sem=send_sems.at[slot],
                recv_sem=recv_sems.at[1 - slot],
                device_id=(right,),
                device_id_type=pl.DeviceIdType.MESH,
            )
            rdma.start()

        wq_g = comm_ref[slot, pl.ds(0, D), :]
        wo_g = comm_ref[slot, pl.ds(D, D), :]
        q_ref[...] = jax.lax.dot(
            x_ref[...], wq_g, preferred_element_type=jnp.float32
        ).astype(jnp.bfloat16)

        for c in kv:
            c.wait()
        if h < N_DEV - 1:
            kv = start_kv(1 - slot, jnp.mod(my - h - 1, N_DEV))

        def head_body(hh, carry, slot=slot):
            off = hh * DH
            q_h = q_ref[:, pl.ds(off, DH)]
            k_h = kbuf[slot, pl.ds(hh, 1)][0]
            s = lax.dot_general(
                q_h, k_h, (((1,), (1,)), ((), ())),
                preferred_element_type=jnp.float32,
            )
            s = s * SCALE + mask_ref[...]
            m = jnp.max(s, axis=1, keepdims=True)
            w = jnp.exp(s - m)
            w = (w / jnp.sum(w, axis=1, keepdims=True)).astype(jnp.bfloat16)
            v_h = vbuf[slot, pl.ds(hh, 1)][0]
            c = lax.dot_general(
                w, v_h, (((1,), (0,)), ((), ())),
                preferred_element_type=jnp.float32,
            )
            ctx_ref[:, pl.ds(off, DH)] = c.astype(jnp.bfloat16)
            return carry

        lax.fori_loop(0, HEADS, head_body, 0)

        out_ref[0, :, :] = out_ref[0, :, :] + jax.lax.dot(
            ctx_ref[...], wo_g, preferred_element_type=jnp.float32
        )

        if h <= N_DEV - 3:
            pl.semaphore_signal(credit_sem, inc=1, device_id=(left,),
                                device_id_type=pl.DeviceIdType.MESH)
        if rdma is not None:
            rdma.wait()

    @functools.partial(pl.run_scoped, exit_sem=pltpu.SemaphoreType.REGULAR)
    def _(exit_sem):
        for nbr in (left, right):
            pl.semaphore_signal(exit_sem, inc=1, device_id=(nbr,),
                                device_id_type=pl.DeviceIdType.MESH)
        pl.semaphore_wait(exit_sem, 2)


def kernel(x, Wq, K_ext, V_ext, Wo):
    i = lax.axis_index("i")
    xs = x[0].astype(jnp.bfloat16)
    wq = Wq.astype(jnp.bfloat16)
    wo = Wo.astype(jnp.bfloat16)
    k = lax.dynamic_index_in_dim(K_ext, i, 0, keepdims=False)
    v = lax.dynamic_index_in_dim(V_ext, i, 0, keepdims=False)
    kt = jnp.transpose(k, (1, 0, 2)).astype(jnp.bfloat16)
    vt = jnp.transpose(v, (1, 0, 2)).astype(jnp.bfloat16)

    vmem = functools.partial(pl.BlockSpec, memory_space=pltpu.VMEM)
    return pl.pallas_call(
        _body,
        out_shape=jax.ShapeDtypeStruct((1, SQ, D), jnp.float32),
        in_specs=[vmem(), vmem(),
                  pl.BlockSpec(memory_space=pl.ANY),
                  pl.BlockSpec(memory_space=pl.ANY),
                  vmem()],
        out_specs=vmem(),
        scratch_shapes=[
            pltpu.VMEM((2, 2 * D, D), jnp.bfloat16),
            pltpu.VMEM((SQ, HEADS * DH), jnp.bfloat16),
            pltpu.VMEM((SQ, HEADS * DH), jnp.bfloat16),
            pltpu.VMEM((SQ, SKV), jnp.float32),
            pltpu.VMEM((2, HEADS, SKV, DH), jnp.bfloat16),
            pltpu.VMEM((2, HEADS, SKV, DH), jnp.bfloat16),
            pltpu.SemaphoreType.DMA((2,)),
            pltpu.SemaphoreType.DMA((2,)),
            pltpu.SemaphoreType.DMA((2, 2)),
            pltpu.SemaphoreType.REGULAR,
        ],
        compiler_params=pltpu.CompilerParams(
            collective_id=0,
            vmem_limit_bytes=128 * 1024 * 1024,
        ),
    )(xs, wq, kt, vt, wo)


# device time: 447314 ns/iter; 1.0276x vs baseline; 1.0276x over previous
import functools
import os

import jax

os.makedirs("/tmp/jax_cache", exist_ok=True)
jax.config.update("jax_compilation_cache_dir", "/tmp/jax_cache")
jax.config.update("jax_persistent_cache_min_compile_time_secs", 0.0)
jax.config.update("jax_persistent_cache_min_entry_size_bytes", -1)

import jax.numpy as jnp
from jax import lax
from jax.experimental import pallas as pl
from jax.experimental.pallas import tpu as pltpu

N_DEV = 8
HEADS = 8
DH = 128
SQ = 1024
SKV = 1024
D = 1024
SCALE = 0.08838834764831843
NEG = -1e9


def _body(x_ref, wq_ref, k_ref, v_ref, wo_ref, out_ref,
          comm_ref, q_ref, ctx_ref, mask_ref, kbuf, vbuf,
          send_sems, recv_sems, kv_sems, credit_sem):
    my = lax.axis_index("i")
    left = jnp.mod(my - 1, N_DEV)
    right = jnp.mod(my + 1, N_DEV)

    def start_kv(slot, g):
        kc = pltpu.make_async_copy(
            k_ref.at[pl.ds(g * HEADS, HEADS)], kbuf.at[slot],
            kv_sems.at[slot, 0])
        vc = pltpu.make_async_copy(
            v_ref.at[pl.ds(g * HEADS, HEADS)], vbuf.at[slot],
            kv_sems.at[slot, 1])
        kc.start()
        vc.start()
        return kc, vc

    kv = start_kv(0, my)

    barrier = pltpu.get_barrier_semaphore()
    for nbr in (left, right):
        pl.semaphore_signal(barrier, inc=1, device_id=(nbr,),
                            device_id_type=pl.DeviceIdType.MESH)
    pl.semaphore_wait(barrier, 2)

    qb = lax.broadcasted_iota(jnp.int32, (SQ, SKV), 0) // 64
    kb = lax.broadcasted_iota(jnp.int32, (SQ, SKV), 1) // 64
    keep = (qb == kb) | (kb == 0) | (jnp.mod(qb + kb, 3) == 0)
    mask_ref[...] = jnp.where(keep, 0.0, NEG).astype(jnp.float32)

    comm_ref[0, pl.ds(0, D), :] = wq_ref[...]
    comm_ref[0, pl.ds(D, D), :] = wo_ref[...]
    out_ref[0, :, :] = jnp.zeros((SQ, D), jnp.float32)

    for h in range(N_DEV):
        slot = h % 2
        g = jnp.mod(my - h, N_DEV)

        rdma = None
        if h < N_DEV - 1:
            if h >= 1:
                pl.semaphore_wait(credit_sem, 1)
            rdma = pltpu.make_async_remote_copy(
                src_ref=comm_ref.at[slot],
                dst_ref=comm_ref.at[1 - slot],
                send_sem=send_sems.at[slot],
                recv_sem=recv_sems.at[1 - slot],
                device_id=(right,),
                device_id_type=pl.DeviceIdType.MESH,
            )
            rdma.start()

        wq_g = comm_ref[slot, pl.ds(0, D), :]
        wo_g = comm_ref[slot, pl.ds(D, D), :]
        q_ref[...] = jax.lax.dot(
            x_ref[...], wq_g, preferred_element_type=jnp.float32
        ).astype(jnp.bfloat16)

        for c in kv:
            c.wait()
        if h < N_DEV - 1:
            kv = start_kv(1 - slot, jnp.mod(my - h - 1, N_DEV))

        def head_body(hh, carry, slot=slot):
            off = hh * DH
            q_h = q_ref[:, pl.ds(off, DH)]
            k_h = kbuf[slot, pl.ds(hh, 1)][0]
            s = lax.dot_general(
                q_h, k_h, (((1,), (1,)), ((), ())),
                preferred_element_type=jnp.float32,
            )
            w = jnp.exp(s * SCALE + mask_ref[...])
            r = 1.0 / jnp.sum(w, axis=1, keepdims=True)
            v_h = vbuf[slot, pl.ds(hh, 1)][0]
            c = lax.dot_general(
                w.astype(jnp.bfloat16), v_h, (((1,), (0,)), ((), ())),
                preferred_element_type=jnp.float32,
            )
            ctx_ref[:, pl.ds(off, DH)] = (c * r).astype(jnp.bfloat16)
            return carry

        lax.fori_loop(0, HEADS, head_body, 0)

        out_ref[0, :, :] = out_ref[0, :, :] + jax.lax.dot(
            ctx_ref[...], wo_g, preferred_element_type=jnp.float32
        )

        if h <= N_DEV - 3:
            pl.semaphore_signal(credit_sem, inc=1, device_id=(left,),
                                device_id_type=pl.DeviceIdType.MESH)
        if rdma is not None:
            rdma.wait()

    @functools.partial(pl.run_scoped, exit_sem=pltpu.SemaphoreType.REGULAR)
    def _(exit_sem):
        for nbr in (left, right):
            pl.semaphore_signal(exit_sem, inc=1, device_id=(nbr,),
                                device_id_type=pl.DeviceIdType.MESH)
        pl.semaphore_wait(exit_sem, 2)


def kernel(x, Wq, K_ext, V_ext, Wo):
    i = lax.axis_index("i")
    xs = x[0].astype(jnp.bfloat16)
    wq = Wq.astype(jnp.bfloat16)
    wo = Wo.astype(jnp.bfloat16)
    k = lax.dynamic_index_in_dim(K_ext, i, 0, keepdims=False)
    v = lax.dynamic_index_in_dim(V_ext, i, 0, keepdims=False)
    kt = jnp.transpose(k, (1, 0, 2)).astype(jnp.bfloat16)
    vt = jnp.transpose(v, (1, 0, 2)).astype(jnp.bfloat16)

    vmem = functools.partial(pl.BlockSpec, memory_space=pltpu.VMEM)
    return pl.pallas_call(
        _body,
        out_shape=jax.ShapeDtypeStruct((1, SQ, D), jnp.float32),
        in_specs=[vmem(), vmem(),
                  pl.BlockSpec(memory_space=pl.ANY),
                  pl.BlockSpec(memory_space=pl.ANY),
                  vmem()],
        out_specs=vmem(),
        scratch_shapes=[
            pltpu.VMEM((2, 2 * D, D), jnp.bfloat16),
            pltpu.VMEM((SQ, HEADS * DH), jnp.bfloat16),
            pltpu.VMEM((SQ, HEADS * DH), jnp.bfloat16),
            pltpu.VMEM((SQ, SKV), jnp.float32),
            pltpu.VMEM((2, HEADS, SKV, DH), jnp.bfloat16),
            pltpu.VMEM((2, HEADS, SKV, DH), jnp.bfloat16),
            pltpu.SemaphoreType.DMA((2,)),
            pltpu.SemaphoreType.DMA((2,)),
            pltpu.SemaphoreType.DMA((2, 2)),
            pltpu.SemaphoreType.REGULAR,
        ],
        compiler_params=pltpu.CompilerParams(
            collective_id=0,
            vmem_limit_bytes=128 * 1024 * 1024,
        ),
    )(xs, wq, kt, vt, wo)
